# Initial kernel scaffold; baseline (speedup 1.0000x reference)
import jax
import jax.numpy as jnp
from jax.experimental import pallas as pl


def kernel(x, edge_index, W1l, W1n, b1, Wbl, Wbn, bb, W2l, W2n, b2, Wgl, Wgn, bg):
    raise NotImplementedError("write your pallas kernel here")


# speedup vs baseline: 8.2185x; 8.2185x over previous
import functools

import jax
import jax.numpy as jnp
from jax import lax
from jax.experimental import pallas as pl
from jax.experimental.pallas import tpu as pltpu
from jax.experimental.pallas import tpu_sc as plsc

N = 50000
E = 800000
F_IN = 192
H = 64
LH = 128
HW = 32
NS = 16
NC = 2
CHUNK = 128
EDGES_PER_TILE = E // NS
NCHUNK = (EDGES_PER_TILE + CHUNK - 1) // CHUNK + 1
EDGES_PAD = NCHUNK * CHUNK
PADE = EDGES_PAD - EDGES_PER_TILE
ROWS_PER_TILE = 3128
ACC_ROWS = NS * ROWS_PER_TILE
BR = 2000


def _agg_body(y0, y1, ep, zrows, a0, a1,
              pbuf, si0, di0, si1, di1, gb0, gb1, acc,
              sem_p0, sem_p1, sem_p2, sem_p3, sem_z, sem_g0, sem_g1):
    c = lax.axis_index("c")
    s = lax.axis_index("s")
    base = s * NCHUNK
    psems = (sem_p0, sem_p1, sem_p2, sem_p3)

    def idx_issue(m, r):
        pltpu.async_copy(ep.at[base + m], pbuf.at[r], psems[r])

    def idx_wait(m, r):
        pltpu.make_async_copy(ep.at[base], pbuf.at[r], psems[r]).wait()

    def unpack(slot, sidx, didx):
        for v in range(CHUNK // 16):
            p = pbuf[slot, pl.ds(16 * v, 16)]
            sidx[0, pl.ds(16 * v, 16)] = p & 0xFFFF
            didx[0, pl.ds(16 * v, 16)] = lax.shift_right_logical(p, 16)

    def run(y, aout):
        gbufs = (gb0, gb1)
        sidxs = (si0, si1)
        didxs = (di0, di1)
        sems = (sem_g0, sem_g1)

        def issue(b):
            pltpu.async_copy(y.at[sidxs[b].at[0]], gbufs[b], sems[b])

        def gwait(b):
            pltpu.make_async_copy(y.at[sidxs[b].at[0]], gbufs[b],
                                  sems[b]).wait()

        cp_z = pltpu.async_copy(
            zrows, acc.at[pl.ds(s * ROWS_PER_TILE, ROWS_PER_TILE)], sem_z)
        for m in range(4):
            idx_issue(m, m)
        idx_wait(0, 0)
        unpack(0, si0, di0)
        issue(0)
        cp_z.wait()
        plsc.subcore_barrier()

        def body(g, carry):
            j0 = 4 * g
            for u in range(4):
                ju = j0 + u
                jn = ju + 1

                @pl.when(jn < NCHUNK)
                def _():
                    idx_wait(jn, (u + 1) % 4)
                    unpack((u + 1) % 4, sidxs[(u + 1) % 2], didxs[(u + 1) % 2])
                    issue((u + 1) % 2)

                @pl.when(jn + 3 < NCHUNK)
                def _():
                    idx_issue(jn + 3, u)

                gwait(u % 2)
                pltpu.sync_copy(gbufs[u % 2], acc.at[didxs[u % 2].at[0]],
                                add=True)
            return carry

        lax.fori_loop(0, NCHUNK // 4, body, 0)
        plsc.subcore_barrier()

        tail = N - (NS - 1) * ROWS_PER_TILE

        @pl.when(s < NS - 1)
        def _():
            pltpu.sync_copy(
                acc.at[pl.ds(s * ROWS_PER_TILE, ROWS_PER_TILE)],
                aout.at[pl.ds(s * ROWS_PER_TILE, ROWS_PER_TILE)])

        @pl.when(s == NS - 1)
        def _():
            pltpu.sync_copy(
                acc.at[pl.ds((NS - 1) * ROWS_PER_TILE, tail)],
                aout.at[pl.ds((NS - 1) * ROWS_PER_TILE, tail)])

    @pl.when(c == 0)
    def _():
        run(y0, a0)

    @pl.when(c == 1)
    def _():
        run(y1, a1)


@functools.cache
def _get_agg():
  return pl.kernel(
    _agg_body,
    out_type=(jax.ShapeDtypeStruct((N, HW), jnp.float32),
              jax.ShapeDtypeStruct((N, HW), jnp.float32)),
    mesh=plsc.VectorSubcoreMesh(core_axis_name="c", subcore_axis_name="s",
                                num_cores=NC, num_subcores=NS),
    compiler_params=pltpu.CompilerParams(use_tc_tiling_on_sc=False),
    scratch_types=[
        pltpu.VMEM((4, CHUNK), jnp.int32),
        pltpu.VMEM((1, CHUNK), jnp.int32),
        pltpu.VMEM((1, CHUNK), jnp.int32),
        pltpu.VMEM((1, CHUNK), jnp.int32),
        pltpu.VMEM((1, CHUNK), jnp.int32),
        pltpu.VMEM((CHUNK, HW), jnp.float32),
        pltpu.VMEM((CHUNK, HW), jnp.float32),
        pltpu.VMEM_SHARED((ACC_ROWS, HW), jnp.float32),
        pltpu.SemaphoreType.DMA,
        pltpu.SemaphoreType.DMA,
        pltpu.SemaphoreType.DMA,
        pltpu.SemaphoreType.DMA,
        pltpu.SemaphoreType.DMA,
        pltpu.SemaphoreType.DMA,
        pltpu.SemaphoreType.DMA,
    ],
  )


_DN = (((1,), (0,)), ((), ()))
_mm = functools.partial(lax.dot_general, dimension_numbers=_DN,
                        preferred_element_type=jnp.float32,
                        precision=lax.Precision.HIGHEST)

_hspec = pl.BlockSpec((BR, HW), lambda i: (i, 0))


def _entry_body(x_ref, w1n_ref, w1l_ref, y0_ref, y1_ref, xl_ref):
    xb = x_ref[...]
    sup = _mm(xb, w1n_ref[...])
    y0_ref[...] = sup[:, :HW]
    y1_ref[...] = sup[:, HW:]
    xl_ref[...] = _mm(xb, w1l_ref[...])


_entry = pl.pallas_call(
    _entry_body,
    grid=(N // BR,),
    in_specs=[
        pl.BlockSpec((BR, F_IN), lambda i: (i, 0)),
        pl.BlockSpec((F_IN, H), lambda i: (0, 0)),
        pl.BlockSpec((F_IN, H), lambda i: (0, 0)),
    ],
    out_specs=(_hspec, _hspec, pl.BlockSpec((BR, H), lambda i: (i, 0))),
    out_shape=(
        jax.ShapeDtypeStruct((N, HW), jnp.float32),
        jax.ShapeDtypeStruct((N, HW), jnp.float32),
        jax.ShapeDtypeStruct((N, H), jnp.float32),
    ),
)


def _combine_entry_body(xl_ref, a0_ref, a1_ref, b_ref, h0_ref, h1_ref):
    agg = jnp.concatenate([a0_ref[...], a1_ref[...]], axis=1)
    z = jnp.maximum(xl_ref[...] + agg + b_ref[...], 0.0)
    h0_ref[...] = z[:, :HW]
    h1_ref[...] = z[:, HW:]


_combine_entry = pl.pallas_call(
    _combine_entry_body,
    grid=(N // BR,),
    in_specs=[pl.BlockSpec((BR, H), lambda i: (i, 0)), _hspec, _hspec,
              pl.BlockSpec((1, H), lambda i: (0, 0))],
    out_specs=(_hspec, _hspec),
    out_shape=(
        jax.ShapeDtypeStruct((N, HW), jnp.float32),
        jax.ShapeDtypeStruct((N, HW), jnp.float32),
    ),
)


def _conv_res_body(h0, h1, a0, a1, w_ref, b_ref, r0, r1, o0, o1):
    hh = jnp.concatenate([h0[...], h1[...], a0[...], a1[...]], axis=1)
    z = jnp.maximum(_mm(hh, w_ref[...]) + b_ref[...], 0.0)
    z = (jnp.concatenate([r0[...], r1[...]], axis=1) + z) * 0.5
    o0[...] = z[:, :HW]
    o1[...] = z[:, HW:]


def _conv_body(h0, h1, a0, a1, w_ref, b_ref, o0, o1):
    hh = jnp.concatenate([h0[...], h1[...], a0[...], a1[...]], axis=1)
    z = jnp.maximum(_mm(hh, w_ref[...]) + b_ref[...], 0.0)
    o0[...] = z[:, :HW]
    o1[...] = z[:, HW:]


def _make_conv(residual):
    body = _conv_res_body if residual else _conv_body
    n_res = 2 if residual else 0
    in_specs = [_hspec] * 4 + [
        pl.BlockSpec((2 * H, H), lambda i: (0, 0)),
        pl.BlockSpec((1, H), lambda i: (0, 0)),
    ] + [_hspec] * n_res
    return pl.pallas_call(
        body,
        grid=(N // BR,),
        in_specs=in_specs,
        out_specs=(_hspec, _hspec),
        out_shape=(
            jax.ShapeDtypeStruct((N, HW), jnp.float32),
            jax.ShapeDtypeStruct((N, HW), jnp.float32),
        ),
    )


_conv_plain = _make_conv(False)
_conv_res = _make_conv(True)


def _exit_body(h0, h1, a0, a1, w2_ref, b2_ref, wgn_ref, wgl_ref,
               s0_ref, s1_ref, xlf_ref):
    hh = jnp.concatenate([h0[...], h1[...], a0[...], a1[...]], axis=1)
    x3 = jnp.maximum(_mm(hh, w2_ref[...]) + b2_ref[...], 0.0)
    sup = _mm(x3, wgn_ref[...])
    s0_ref[...] = sup[:, :HW]
    s1_ref[...] = sup[:, HW:]
    xlf_ref[...] = _mm(x3, wgl_ref[...])


_exit = pl.pallas_call(
    _exit_body,
    grid=(N // BR,),
    in_specs=[_hspec] * 4 + [
        pl.BlockSpec((2 * H, LH), lambda i: (0, 0)),
        pl.BlockSpec((1, LH), lambda i: (0, 0)),
        pl.BlockSpec((LH, H), lambda i: (0, 0)),
        pl.BlockSpec((LH, 8), lambda i: (0, 0)),
    ],
    out_specs=(_hspec, _hspec, pl.BlockSpec((BR, 8), lambda i: (i, 0))),
    out_shape=(
        jax.ShapeDtypeStruct((N, HW), jnp.float32),
        jax.ShapeDtypeStruct((N, HW), jnp.float32),
        jax.ShapeDtypeStruct((N, 8), jnp.float32),
    ),
)


def _final_body(xlf_ref, af0_ref, bg_ref, o_ref):
    z = xlf_ref[...] + af0_ref[...][:, :8] + bg_ref[...]
    o_ref[...] = z[:, :3]


_final = pl.pallas_call(
    _final_body,
    grid=(N // BR,),
    in_specs=[
        pl.BlockSpec((BR, 8), lambda i: (i, 0)),
        _hspec,
        pl.BlockSpec((1, 8), lambda i: (0, 0)),
    ],
    out_specs=pl.BlockSpec((BR, 3), lambda i: (i, 0)),
    out_shape=jax.ShapeDtypeStruct((N, 3), jnp.float32),
)


def kernel(x, edge_index, W1l, W1n, b1, Wbl, Wbn, bb, W2l, W2n, b2,
           Wgl, Wgn, bg):
    src = edge_index[0]
    dst = edge_index[1]

    pad_src = jnp.broadcast_to(
        (jnp.arange(PADE, dtype=jnp.int32) * 521) % N, (NS, PADE))
    pad_dst = jnp.broadcast_to(
        N + (jnp.arange(PADE, dtype=jnp.int32) % (ACC_ROWS - N)), (NS, PADE))
    srcp = jnp.concatenate(
        [src.reshape(NS, EDGES_PER_TILE), pad_src], axis=1)
    dstp = jnp.concatenate(
        [dst.reshape(NS, EDGES_PER_TILE), pad_dst], axis=1)
    ep = (jnp.left_shift(dstp, 16) | srcp).reshape(NS * NCHUNK, CHUNK)
    ep = jnp.concatenate(
        [ep, jnp.zeros((3 * NS * NCHUNK, CHUNK), jnp.int32)], axis=0)
    zrows = jnp.zeros((ROWS_PER_TILE, HW), jnp.float32)

    _agg = _get_agg()
    agg = lambda u: _agg(*u, ep, zrows)

    b1r = b1.reshape(1, H)
    bbr = bb.reshape(2 * 6, 1, H)
    wcat = jnp.concatenate([Wbl, Wbn], axis=1)
    w2cat = jnp.concatenate([W2l, W2n], axis=0)
    b2r = b2.reshape(1, LH)
    wgn_pad = jnp.pad(Wgn, ((0, 0), (0, H - 3)))
    wgl_pad = jnp.pad(Wgl, ((0, 0), (0, 8 - 3)))
    bgp = jnp.pad(bg.reshape(1, 3), ((0, 0), (0, 5)))

    *y, xl = _entry(x, W1n, W1l)
    a = agg(y)
    h = _combine_entry(xl, *a, b1r)

    for i in range(6):
        a = agg(h)
        z = _conv_plain(*h, *a, wcat[2 * i], bbr[2 * i])
        a = agg(z)
        h = _conv_res(*z, *a, wcat[2 * i + 1], bbr[2 * i + 1], *h)

    a = agg(h)
    *s, xlf = _exit(*h, *a, w2cat, b2r, wgn_pad, wgl_pad)

    af = agg(s)
    return _final(xlf, af[0], bgp)


# speedup vs baseline: 10.9941x; 1.3377x over previous
import functools

import jax
import jax.numpy as jnp
from jax import lax
from jax.experimental import pallas as pl
from jax.experimental.pallas import tpu as pltpu
from jax.experimental.pallas import tpu_sc as plsc

N = 50000
E = 800000
F_IN = 192
H = 64
LH = 128
HW = 32
NS = 16
NC = 2
CHUNK = 128
EDGES_PER_TILE = E // NS
NCHUNK = (EDGES_PER_TILE + CHUNK - 1) // CHUNK + 1
EDGES_PAD = NCHUNK * CHUNK
PADE = EDGES_PAD - EDGES_PER_TILE
ROWS_PER_TILE = 3128
ACC_ROWS = NS * ROWS_PER_TILE
BR = 5000


def _agg_body(y0, y1, ep, zrows, a0, a1,
              pbuf, si0, si1, si2, si3, di0, di1, di2, di3,
              gb0, gb1, gb2, gb3, acc,
              sem_p0, sem_p1, sem_p2, sem_p3, sem_z,
              sem_g0, sem_g1, sem_g2, sem_g3,
              sem_s0, sem_s1, sem_s2, sem_s3):
    c = lax.axis_index("c")
    s = lax.axis_index("s")
    base = s * NCHUNK
    psems = (sem_p0, sem_p1, sem_p2, sem_p3)

    def idx_issue(m, r):
        pltpu.async_copy(ep.at[base + m], pbuf.at[r], psems[r])

    def idx_wait(m, r):
        pltpu.make_async_copy(ep.at[base], pbuf.at[r], psems[r]).wait()

    def unpack(slot, sidx, didx):
        for v in range(CHUNK // 16):
            p = pbuf[slot, pl.ds(16 * v, 16)]
            sidx[0, pl.ds(16 * v, 16)] = p & 0xFFFF
            didx[0, pl.ds(16 * v, 16)] = lax.shift_right_logical(p, 16)

    def run(y, aout):
        gbufs = (gb0, gb1, gb2, gb3)
        sidxs = (si0, si1, si2, si3)
        didxs = (di0, di1, di2, di3)
        gsems = (sem_g0, sem_g1, sem_g2, sem_g3)
        ssems = (sem_s0, sem_s1, sem_s2, sem_s3)

        def issue(b):
            pltpu.async_copy(y.at[sidxs[b].at[0]], gbufs[b], gsems[b])

        def gwait(b):
            pltpu.make_async_copy(y.at[sidxs[b].at[0]], gbufs[b],
                                  gsems[b]).wait()

        def scat(b):
            pltpu.async_copy(gbufs[b], acc.at[didxs[b].at[0]], ssems[b],
                             add=True)

        def swait(b):
            pltpu.make_async_copy(gbufs[b], acc.at[didxs[b].at[0]],
                                  ssems[b]).wait()

        cp_z = pltpu.async_copy(
            zrows, acc.at[pl.ds(s * ROWS_PER_TILE, ROWS_PER_TILE)], sem_z)
        for m in range(4):
            idx_issue(m, m)
        idx_wait(0, 0)
        unpack(0, si0, di0)
        issue(0)
        idx_wait(1, 1)
        unpack(1, si1, di1)
        issue(1)
        cp_z.wait()
        plsc.subcore_barrier()

        def body(g, carry):
            j0 = 4 * g
            for u in range(4):
                ju = j0 + u
                jn = ju + 2
                sn = (u + 2) % 4

                @pl.when(jn < NCHUNK)
                def _():
                    idx_wait(jn, sn)

                @pl.when(jnp.logical_and(jn >= 4, jn < NCHUNK))
                def _():
                    swait(sn)

                @pl.when(jn < NCHUNK)
                def _():
                    unpack(sn, sidxs[sn], didxs[sn])
                    issue(sn)

                @pl.when(ju + 4 < NCHUNK)
                def _():
                    idx_issue(ju + 4, u)

                gwait(u)
                scat(u)
            return carry

        lax.fori_loop(0, NCHUNK // 4, body, 0)
        for u in range(4):
            swait(u)
        plsc.subcore_barrier()

        tail = N - (NS - 1) * ROWS_PER_TILE

        @pl.when(s < NS - 1)
        def _():
            pltpu.sync_copy(
                acc.at[pl.ds(s * ROWS_PER_TILE, ROWS_PER_TILE)],
                aout.at[pl.ds(s * ROWS_PER_TILE, ROWS_PER_TILE)])

        @pl.when(s == NS - 1)
        def _():
            pltpu.sync_copy(
                acc.at[pl.ds((NS - 1) * ROWS_PER_TILE, tail)],
                aout.at[pl.ds((NS - 1) * ROWS_PER_TILE, tail)])

    @pl.when(c == 0)
    def _():
        run(y0, a0)

    @pl.when(c == 1)
    def _():
        run(y1, a1)


@functools.cache
def _get_agg():
  return pl.kernel(
    _agg_body,
    out_type=(jax.ShapeDtypeStruct((N, HW), jnp.float32),
              jax.ShapeDtypeStruct((N, HW), jnp.float32)),
    mesh=plsc.VectorSubcoreMesh(core_axis_name="c", subcore_axis_name="s",
                                num_cores=NC, num_subcores=NS),
    compiler_params=pltpu.CompilerParams(use_tc_tiling_on_sc=False),
    scratch_types=[pltpu.VMEM((4, CHUNK), jnp.int32)]
    + [pltpu.VMEM((1, CHUNK), jnp.int32)] * 8
    + [pltpu.VMEM((CHUNK, HW), jnp.float32)] * 4
    + [pltpu.VMEM_SHARED((ACC_ROWS, HW), jnp.float32)]
    + [pltpu.SemaphoreType.DMA] * 13,
  )


_DN = (((1,), (0,)), ((), ()))
_mm = functools.partial(lax.dot_general, dimension_numbers=_DN,
                        preferred_element_type=jnp.float32)

_hspec = pl.BlockSpec((BR, HW), lambda i: (i, 0))


def _entry_body(x_ref, w1n_ref, w1l_ref, y0_ref, y1_ref, xl_ref):
    xb = x_ref[...]
    sup = _mm(xb, w1n_ref[...])
    y0_ref[...] = sup[:, :HW]
    y1_ref[...] = sup[:, HW:]
    xl_ref[...] = _mm(xb, w1l_ref[...])


_entry = pl.pallas_call(
    _entry_body,
    grid=(N // BR,),
    in_specs=[
        pl.BlockSpec((BR, F_IN), lambda i: (i, 0)),
        pl.BlockSpec((F_IN, H), lambda i: (0, 0)),
        pl.BlockSpec((F_IN, H), lambda i: (0, 0)),
    ],
    out_specs=(_hspec, _hspec, pl.BlockSpec((BR, H), lambda i: (i, 0))),
    out_shape=(
        jax.ShapeDtypeStruct((N, HW), jnp.float32),
        jax.ShapeDtypeStruct((N, HW), jnp.float32),
        jax.ShapeDtypeStruct((N, H), jnp.float32),
    ),
)


def _combine_entry_body(xl_ref, a0_ref, a1_ref, b_ref, h0_ref, h1_ref):
    agg = jnp.concatenate([a0_ref[...], a1_ref[...]], axis=1)
    z = jnp.maximum(xl_ref[...] + agg + b_ref[...], 0.0)
    h0_ref[...] = z[:, :HW]
    h1_ref[...] = z[:, HW:]


_combine_entry = pl.pallas_call(
    _combine_entry_body,
    grid=(N // BR,),
    in_specs=[pl.BlockSpec((BR, H), lambda i: (i, 0)), _hspec, _hspec,
              pl.BlockSpec((1, H), lambda i: (0, 0))],
    out_specs=(_hspec, _hspec),
    out_shape=(
        jax.ShapeDtypeStruct((N, HW), jnp.float32),
        jax.ShapeDtypeStruct((N, HW), jnp.float32),
    ),
)


def _conv_res_body(h0, h1, a0, a1, w_ref, b_ref, r0, r1, o0, o1):
    hh = jnp.concatenate([h0[...], h1[...], a0[...], a1[...]], axis=1)
    z = jnp.maximum(_mm(hh, w_ref[...]) + b_ref[...], 0.0)
    z = (jnp.concatenate([r0[...], r1[...]], axis=1) + z) * 0.5
    o0[...] = z[:, :HW]
    o1[...] = z[:, HW:]


def _conv_body(h0, h1, a0, a1, w_ref, b_ref, o0, o1):
    hh = jnp.concatenate([h0[...], h1[...], a0[...], a1[...]], axis=1)
    z = jnp.maximum(_mm(hh, w_ref[...]) + b_ref[...], 0.0)
    o0[...] = z[:, :HW]
    o1[...] = z[:, HW:]


def _make_conv(residual):
    body = _conv_res_body if residual else _conv_body
    n_res = 2 if residual else 0
    in_specs = [_hspec] * 4 + [
        pl.BlockSpec((2 * H, H), lambda i: (0, 0)),
        pl.BlockSpec((1, H), lambda i: (0, 0)),
    ] + [_hspec] * n_res
    return pl.pallas_call(
        body,
        grid=(N // BR,),
        in_specs=in_specs,
        out_specs=(_hspec, _hspec),
        out_shape=(
            jax.ShapeDtypeStruct((N, HW), jnp.float32),
            jax.ShapeDtypeStruct((N, HW), jnp.float32),
        ),
    )


_conv_plain = _make_conv(False)
_conv_res = _make_conv(True)


def _exit_body(h0, h1, a0, a1, w2_ref, b2_ref, wgn_ref, wgl_ref,
               s0_ref, s1_ref, xlf_ref):
    hh = jnp.concatenate([h0[...], h1[...], a0[...], a1[...]], axis=1)
    x3 = jnp.maximum(_mm(hh, w2_ref[...]) + b2_ref[...], 0.0)
    sup = _mm(x3, wgn_ref[...])
    s0_ref[...] = sup[:, :HW]
    s1_ref[...] = sup[:, HW:]
    xlf_ref[...] = _mm(x3, wgl_ref[...])


_exit = pl.pallas_call(
    _exit_body,
    grid=(N // BR,),
    in_specs=[_hspec] * 4 + [
        pl.BlockSpec((2 * H, LH), lambda i: (0, 0)),
        pl.BlockSpec((1, LH), lambda i: (0, 0)),
        pl.BlockSpec((LH, H), lambda i: (0, 0)),
        pl.BlockSpec((LH, 8), lambda i: (0, 0)),
    ],
    out_specs=(_hspec, _hspec, pl.BlockSpec((BR, 8), lambda i: (i, 0))),
    out_shape=(
        jax.ShapeDtypeStruct((N, HW), jnp.float32),
        jax.ShapeDtypeStruct((N, HW), jnp.float32),
        jax.ShapeDtypeStruct((N, 8), jnp.float32),
    ),
)


def _final_body(xlf_ref, af0_ref, bg_ref, o_ref):
    z = xlf_ref[...] + af0_ref[...][:, :8] + bg_ref[...]
    o_ref[...] = z[:, :3]


_final = pl.pallas_call(
    _final_body,
    grid=(N // BR,),
    in_specs=[
        pl.BlockSpec((BR, 8), lambda i: (i, 0)),
        _hspec,
        pl.BlockSpec((1, 8), lambda i: (0, 0)),
    ],
    out_specs=pl.BlockSpec((BR, 3), lambda i: (i, 0)),
    out_shape=jax.ShapeDtypeStruct((N, 3), jnp.float32),
)


def kernel(x, edge_index, W1l, W1n, b1, Wbl, Wbn, bb, W2l, W2n, b2,
           Wgl, Wgn, bg):
    src = edge_index[0]
    dst = edge_index[1]

    pad_src = jnp.broadcast_to(
        (jnp.arange(PADE, dtype=jnp.int32) * 521) % N, (NS, PADE))
    pad_dst = jnp.broadcast_to(
        N + (jnp.arange(PADE, dtype=jnp.int32) % (ACC_ROWS - N)), (NS, PADE))
    srcp = jnp.concatenate(
        [src.reshape(NS, EDGES_PER_TILE), pad_src], axis=1)
    dstp = jnp.concatenate(
        [dst.reshape(NS, EDGES_PER_TILE), pad_dst], axis=1)
    ep = (jnp.left_shift(dstp, 16) | srcp).reshape(NS * NCHUNK, CHUNK)
    ep = jnp.concatenate(
        [ep, jnp.zeros((3 * NS * NCHUNK, CHUNK), jnp.int32)], axis=0)
    zrows = jnp.zeros((ROWS_PER_TILE, HW), jnp.float32)

    _agg = _get_agg()
    agg = lambda u: _agg(*u, ep, zrows)

    b1r = b1.reshape(1, H)
    bbr = bb.reshape(2 * 6, 1, H)
    wcat = jnp.concatenate([Wbl, Wbn], axis=1)
    w2cat = jnp.concatenate([W2l, W2n], axis=0)
    b2r = b2.reshape(1, LH)
    wgn_pad = jnp.pad(Wgn, ((0, 0), (0, H - 3)))
    wgl_pad = jnp.pad(Wgl, ((0, 0), (0, 8 - 3)))
    bgp = jnp.pad(bg.reshape(1, 3), ((0, 0), (0, 5)))

    *y, xl = _entry(x, W1n, W1l)
    a = agg(y)
    h = _combine_entry(xl, *a, b1r)

    for i in range(6):
        a = agg(h)
        z = _conv_plain(*h, *a, wcat[2 * i], bbr[2 * i])
        a = agg(z)
        h = _conv_res(*z, *a, wcat[2 * i + 1], bbr[2 * i + 1], *h)

    a = agg(h)
    *s, xlf = _exit(*h, *a, w2cat, b2r, wgn_pad, wgl_pad)

    af = agg(s)
    return _final(xlf, af[0], bgp)
